# baseline (speedup 1.0000x reference)
import numpy as np

import jax
import jax.numpy as jnp
from jax.experimental import pallas as pl
from jax.sharding import Mesh, PartitionSpec

try:
    _shard_map = jax.shard_map
except AttributeError:
    from jax.experimental.shard_map import shard_map as _shard_map

_D = 3
_N = 4096
_R = 512


def _knn_mean_body(x_ref, out_ref):
    rb = pl.program_id(1)
    X = x_ref[0]
    rows = x_ref[0, :, pl.ds(rb * _R, _R)]
    xx = jnp.sum(X * X, axis=0, keepdims=True)
    xx_r = jnp.sum(rows * rows, axis=0)[:, None]
    G = jax.lax.dot_general(
        rows, X, dimension_numbers=(((0,), (0,)), ((), ())),
        precision=jax.lax.Precision.DEFAULT)
    P = 2.0 * G - xx_r - xx

    iota = jax.lax.broadcasted_iota(jnp.int32, (_R, _N), 1)
    W = jnp.zeros((_R, _N), jnp.float32)
    for t in range(5):
        idx = jnp.argmax(P, axis=1)
        onehot = iota == idx[:, None]
        if t > 0:
            W += onehot.astype(jnp.float32)
        P = jnp.where(onehot, -jnp.inf, P)

    M = jax.lax.dot_general(
        X, W, dimension_numbers=(((1,), (1,)), ((), ())),
        precision=jax.lax.Precision.HIGHEST)
    out_ref[0] = M * 0.25


def _neighbor_means(x):
    b = x.shape[0]
    return pl.pallas_call(
        _knn_mean_body,
        grid=(b, _N // _R),
        in_specs=[pl.BlockSpec((1, _D, _N), lambda b, r: (b, 0, 0))],
        out_specs=pl.BlockSpec((1, _D, _R), lambda b, r: (b, 0, r)),
        out_shape=jax.ShapeDtypeStruct((b, _D, _N), jnp.float32),
    )(x)


def _upsample_local(x):
    return jnp.concatenate([x, _neighbor_means(x)], axis=2)


def kernel(x):
    devs = [d for d in jax.devices() if d.platform == "tpu"]
    n_shard = 2 if len(devs) >= 2 and x.shape[0] % 2 == 0 else 1
    if n_shard > 1:
        mesh = Mesh(np.array(devs[:n_shard]), ("d",))
        return _shard_map(
            _upsample_local, mesh=mesh,
            in_specs=PartitionSpec("d"), out_specs=PartitionSpec("d"),
            check_vma=False,
        )(x)
    return _upsample_local(x)


# speedup vs baseline: 1.3028x; 1.3028x over previous
import numpy as np

import jax
import jax.numpy as jnp
from jax.experimental import pallas as pl
from jax.sharding import Mesh, PartitionSpec

try:
    _shard_map = jax.shard_map
except AttributeError:
    from jax.experimental.shard_map import shard_map as _shard_map

_D = 3
_N = 4096
_R = 512


def _knn_mean_body(x_ref, out_ref):
    rb = pl.program_id(1)
    X = x_ref[0]
    rows = x_ref[0, :, pl.ds(rb * _R, _R)]
    xx = jnp.sum(X * X, axis=0, keepdims=True)
    xx_r = jnp.sum(rows * rows, axis=0)[:, None]
    G = jax.lax.dot_general(
        rows, X, dimension_numbers=(((0,), (0,)), ((), ())),
        precision=jax.lax.Precision.DEFAULT)
    P = 2.0 * G - xx_r - xx

    C = 128
    neg = jnp.full((_R, C), -jnp.inf, jnp.float32)
    V = [P[:, 0:C], neg, neg, neg, neg]
    for c in range(1, _N // C):
        cur = P[:, c * C:(c + 1) * C]
        for l in range(5):
            hi = jnp.maximum(V[l], cur)
            cur = jnp.minimum(V[l], cur)
            V[l] = hi

    v1 = jnp.max(V[0], axis=1, keepdims=True)
    iota128 = jax.lax.broadcasted_iota(jnp.int32, (_R, C), 1)
    a0 = jnp.argmax(V[0], axis=1)[:, None]
    V[0] = jnp.where(iota128 == a0, -jnp.inf, V[0])
    Vcat = jnp.concatenate(V, axis=1)
    iota640 = jax.lax.broadcasted_iota(jnp.int32, (_R, 5 * C), 1)
    for _ in range(3):
        a = jnp.argmax(Vcat, axis=1)[:, None]
        Vcat = jnp.where(iota640 == a, -jnp.inf, Vcat)
    v5 = jnp.max(Vcat, axis=1, keepdims=True)

    iota = jax.lax.broadcasted_iota(jnp.int32, (_R, _N), 1)
    ge = (P >= v5).astype(jnp.float32)
    cnt = jnp.sum(ge, axis=1, keepdims=True)
    last_idx = jnp.max(jnp.where(P == v5, iota, -1), axis=1, keepdims=True)
    first1 = jnp.min(jnp.where(P == v1, iota, 1 << 20), axis=1, keepdims=True)
    corr = ((iota == first1).astype(jnp.float32)
            + (iota == last_idx).astype(jnp.float32)
            * (cnt > 5.5).astype(jnp.float32))
    W = ge - corr

    Xh = X.astype(jnp.bfloat16).astype(jnp.float32)
    Xl = X - Xh
    dn = (((1,), (1,)), ((), ()))
    M = (jax.lax.dot_general(Xh, W, dimension_numbers=dn,
                             precision=jax.lax.Precision.DEFAULT)
         + jax.lax.dot_general(Xl, W, dimension_numbers=dn,
                               precision=jax.lax.Precision.DEFAULT))
    out_ref[0] = M * 0.25


def _neighbor_means(x):
    b = x.shape[0]
    return pl.pallas_call(
        _knn_mean_body,
        grid=(b, _N // _R),
        in_specs=[pl.BlockSpec((1, _D, _N), lambda b, r: (b, 0, 0))],
        out_specs=pl.BlockSpec((1, _D, _R), lambda b, r: (b, 0, r)),
        out_shape=jax.ShapeDtypeStruct((b, _D, _N), jnp.float32),
    )(x)


def _upsample_local(x):
    return jnp.concatenate([x, _neighbor_means(x)], axis=2)


def kernel(x):
    devs = [d for d in jax.devices() if d.platform == "tpu"]
    n_shard = 2 if len(devs) >= 2 and x.shape[0] % 2 == 0 else 1
    if n_shard > 1:
        mesh = Mesh(np.array(devs[:n_shard]), ("d",))
        return _shard_map(
            _upsample_local, mesh=mesh,
            in_specs=PartitionSpec("d"), out_specs=PartitionSpec("d"),
            check_vma=False,
        )(x)
    return _upsample_local(x)


# speedup vs baseline: 1.4503x; 1.1133x over previous
import numpy as np

import jax
import jax.numpy as jnp
from jax.experimental import pallas as pl
from jax.sharding import Mesh, PartitionSpec

try:
    _shard_map = jax.shard_map
except AttributeError:
    from jax.experimental.shard_map import shard_map as _shard_map

_D = 3
_N = 4096
_R = 512


def _knn_mean_body(x_ref, out_ref):
    rb = pl.program_id(1)
    X = x_ref[0]
    rows = x_ref[0, :, pl.ds(rb * _R, _R)]
    xx = jnp.sum(X * X, axis=0, keepdims=True)
    xx_r = jnp.sum(rows * rows, axis=0)[:, None]
    G = jax.lax.dot_general(
        rows, X, dimension_numbers=(((0,), (0,)), ((), ())),
        precision=jax.lax.Precision.DEFAULT)
    P = 2.0 * G - xx_r - xx

    C = 128
    neg = jnp.full((_R, C), -jnp.inf, jnp.float32)
    V = [P[:, 0:C], neg, neg, neg, neg]
    for c in range(1, _N // C):
        cur = P[:, c * C:(c + 1) * C]
        for l in range(5):
            hi = jnp.maximum(V[l], cur)
            cur = jnp.minimum(V[l], cur)
            V[l] = hi

    v1 = jnp.max(V[0], axis=1, keepdims=True)
    iota128 = jax.lax.broadcasted_iota(jnp.int32, (_R, C), 1)
    a0 = jnp.argmax(V[0], axis=1)[:, None]
    V[0] = jnp.where(iota128 == a0, -jnp.inf, V[0])
    Vcat = jnp.concatenate(V, axis=1)
    iota640 = jax.lax.broadcasted_iota(jnp.int32, (_R, 5 * C), 1)
    for _ in range(3):
        a = jnp.argmax(Vcat, axis=1)[:, None]
        Vcat = jnp.where(iota640 == a, -jnp.inf, Vcat)
    v5 = jnp.max(Vcat, axis=1, keepdims=True)

    iota = jax.lax.broadcasted_iota(jnp.int32, (_R, _N), 1)
    ge = (P >= v5).astype(jnp.float32)
    cnt = jnp.sum(ge, axis=1, keepdims=True)
    last_idx = jnp.max(jnp.where(P == v5, iota, -1), axis=1, keepdims=True)
    first1 = jnp.min(jnp.where(P == v1, iota, 1 << 20), axis=1, keepdims=True)
    corr = ((iota == first1).astype(jnp.float32)
            + (iota == last_idx).astype(jnp.float32)
            * (cnt > 5.5).astype(jnp.float32))
    W = ge - corr

    Xh = X.astype(jnp.bfloat16).astype(jnp.float32)
    Xl = X - Xh
    dn = (((1,), (1,)), ((), ()))
    M = (jax.lax.dot_general(Xh, W, dimension_numbers=dn,
                             precision=jax.lax.Precision.DEFAULT)
         + jax.lax.dot_general(Xl, W, dimension_numbers=dn,
                               precision=jax.lax.Precision.DEFAULT))
    out_ref[0] = M * 0.25


def _neighbor_means(x):
    b = x.shape[0]
    return pl.pallas_call(
        _knn_mean_body,
        grid=(b, _N // _R),
        in_specs=[pl.BlockSpec((1, _D, _N), lambda b, r: (b, 0, 0))],
        out_specs=pl.BlockSpec((1, _D, _R), lambda b, r: (b, 0, r)),
        out_shape=jax.ShapeDtypeStruct((b, _D, _N), jnp.float32),
    )(x)


def _upsample_local(x):
    return jnp.concatenate([x, _neighbor_means(x)], axis=2)


def kernel(x):
    devs = [d for d in jax.devices() if d.platform == "tpu"]
    n_shard = 1
    if n_shard > 1:
        mesh = Mesh(np.array(devs[:n_shard]), ("d",))
        return _shard_map(
            _upsample_local, mesh=mesh,
            in_specs=PartitionSpec("d"), out_specs=PartitionSpec("d"),
            check_vma=False,
        )(x)
    return _upsample_local(x)


# speedup vs baseline: 1.4525x; 1.0015x over previous
import jax
import jax.numpy as jnp
from jax.experimental import pallas as pl

_D = 3
_N = 4096
_R = 512


def _knn_mean_body(x_ref, out_ref):
    rb = pl.program_id(1)
    X = x_ref[0]
    rows = x_ref[0, :, pl.ds(rb * _R, _R)]
    xx = jnp.sum(X * X, axis=0, keepdims=True)
    xx_r = jnp.sum(rows * rows, axis=0)[:, None]
    G = jax.lax.dot_general(
        rows, X, dimension_numbers=(((0,), (0,)), ((), ())),
        precision=jax.lax.Precision.DEFAULT)
    P = 2.0 * G - xx_r - xx

    C = 128
    neg = jnp.full((_R, C), -jnp.inf, jnp.float32)
    V = [P[:, 0:C], neg, neg, neg, neg]
    for c in range(1, _N // C):
        cur = P[:, c * C:(c + 1) * C]
        for l in range(5):
            hi = jnp.maximum(V[l], cur)
            cur = jnp.minimum(V[l], cur)
            V[l] = hi

    v1 = jnp.max(V[0], axis=1, keepdims=True)
    iota128 = jax.lax.broadcasted_iota(jnp.int32, (_R, C), 1)
    a0 = jnp.argmax(V[0], axis=1)[:, None]
    V[0] = jnp.where(iota128 == a0, -jnp.inf, V[0])
    Vcat = jnp.concatenate(V, axis=1)
    iota640 = jax.lax.broadcasted_iota(jnp.int32, (_R, 5 * C), 1)
    for _ in range(3):
        a = jnp.argmax(Vcat, axis=1)[:, None]
        Vcat = jnp.where(iota640 == a, -jnp.inf, Vcat)
    v5 = jnp.max(Vcat, axis=1, keepdims=True)

    iota = jax.lax.broadcasted_iota(jnp.int32, (_R, _N), 1)
    ge = (P >= v5).astype(jnp.float32)
    cnt = jnp.sum(ge, axis=1, keepdims=True)
    last_idx = jnp.max(jnp.where(P == v5, iota, -1), axis=1, keepdims=True)
    first1 = jnp.min(jnp.where(P == v1, iota, 1 << 20), axis=1, keepdims=True)
    corr = ((iota == first1).astype(jnp.float32)
            + (iota == last_idx).astype(jnp.float32)
            * (cnt > 5.5).astype(jnp.float32))
    W = ge - corr

    Xh = X.astype(jnp.bfloat16).astype(jnp.float32)
    Xl = X - Xh
    dn = (((1,), (1,)), ((), ()))
    M = (jax.lax.dot_general(Xh, W, dimension_numbers=dn,
                             precision=jax.lax.Precision.DEFAULT)
         + jax.lax.dot_general(Xl, W, dimension_numbers=dn,
                               precision=jax.lax.Precision.DEFAULT))
    out_ref[0] = M * 0.25


def _neighbor_means(x):
    b = x.shape[0]
    return pl.pallas_call(
        _knn_mean_body,
        grid=(b, _N // _R),
        in_specs=[pl.BlockSpec((1, _D, _N), lambda b, r: (b, 0, 0))],
        out_specs=pl.BlockSpec((1, _D, _R), lambda b, r: (b, 0, r)),
        out_shape=jax.ShapeDtypeStruct((b, _D, _N), jnp.float32),
    )(x)


def kernel(x):
    return jnp.concatenate([x, _neighbor_means(x)], axis=2)
